# baseline (speedup 1.0000x reference)
import functools

import jax
import jax.numpy as jnp
from jax import lax
from jax.experimental import pallas as pl
from jax.experimental.pallas import tpu as pltpu

_BLK = 1024


def _tc_embed(idx3, table):
    nb = idx3.shape[0]
    vocab, embed_dim = table.shape

    def body(idx_ref, tab_ref, out_ref):
        idx = idx_ref[0].reshape(1, _BLK)
        vio = lax.broadcasted_iota(jnp.int32, (vocab, _BLK), 0)
        oh = (idx == vio).astype(jnp.float32)
        out_ref[...] = lax.dot_general(
            oh, tab_ref[...], (((0,), (0,)), ((), ())),
            preferred_element_type=jnp.float32,
            precision=lax.Precision.HIGHEST)

    return pl.pallas_call(
        body,
        grid=(nb,),
        in_specs=[
            pl.BlockSpec((1, 8, 128), lambda i: (i, 0, 0)),
            pl.BlockSpec((vocab, embed_dim), lambda i: (0, 0)),
        ],
        out_specs=pl.BlockSpec((_BLK, embed_dim), lambda i: (i, 0)),
        out_shape=jax.ShapeDtypeStruct((nb * _BLK, embed_dim), jnp.float32),
    )(idx3, table)


def kernel(indices, table):
    batch, hist = indices.shape
    vocab, embed_dim = table.shape
    n = batch * hist
    idx3 = indices.reshape(n // _BLK, 8, 128)
    out = _tc_embed(idx3, table)
    return out.reshape(batch, hist, embed_dim)


# speedup vs baseline: 1.4187x; 1.4187x over previous
import functools

import jax
import jax.numpy as jnp
from jax import lax
from jax.experimental import pallas as pl
from jax.experimental.pallas import tpu as pltpu

_BLK = 1024


def _tc_embed(idx3, table_hl):
    nb = idx3.shape[0]
    vocab, two_d = table_hl.shape
    embed_dim = two_d // 2

    def body(idx_ref, tab_ref, out_ref):
        idx = idx_ref[0].reshape(1, _BLK)
        vio = lax.broadcasted_iota(jnp.int32, (vocab, _BLK), 0)
        oh = (idx == vio).astype(jnp.bfloat16)
        r = lax.dot_general(oh, tab_ref[...], (((0,), (0,)), ((), ())),
                            preferred_element_type=jnp.float32)
        out_ref[...] = r[:, :embed_dim] + r[:, embed_dim:]

    return pl.pallas_call(
        body,
        grid=(nb,),
        in_specs=[
            pl.BlockSpec((1, 8, 128), lambda i: (i, 0, 0)),
            pl.BlockSpec((vocab, two_d), lambda i: (0, 0)),
        ],
        out_specs=pl.BlockSpec((_BLK, embed_dim), lambda i: (i, 0)),
        out_shape=jax.ShapeDtypeStruct((nb * _BLK, embed_dim), jnp.float32),
    )(idx3, table_hl)


def kernel(indices, table):
    batch, hist = indices.shape
    vocab, embed_dim = table.shape
    n = batch * hist
    th = table.astype(jnp.bfloat16)
    tl = (table - th.astype(jnp.float32)).astype(jnp.bfloat16)
    table_hl = jnp.concatenate([th, tl], axis=1)
    idx3 = indices.reshape(n // _BLK, 8, 128)
    out = _tc_embed(idx3, table_hl)
    return out.reshape(batch, hist, embed_dim)
